# baseline (speedup 1.0000x reference)
import jax
import jax.numpy as jnp
import numpy as np
from jax import lax
from jax.experimental import pallas as pl
from jax.experimental.pallas import tpu as pltpu
from jax.experimental.pallas import tpu_sc as plsc

N_NODES = 10000
N_EDGES = 320000
D = 128

NC = 2
NS = 16
LANES = 16
VPR = D // LANES

EDGES_PER_TILE = N_EDGES // (NC * NS)
CHUNK = 80
NCHUNKS = EDGES_PER_TILE // CHUNK

ACC_ROWS = 10240
ROWS_PER_TILE = ACC_ROWS // NS


def _mul_rows(pr, xgh, w):
    hi_mask = jnp.full((LANES,), -65536, jnp.int32)
    @plsc.parallel_loop(0, CHUNK, step=1, unroll=4)
    def _(e):
        for k in range(D // 32):
            v = xgh[e, pl.ds(k * LANES, LANES)]
            a = (v << 16).view(jnp.float32)
            b = (v & hi_mask).view(jnp.float32)
            sla = pl.ds(k * 32, LANES)
            slb = pl.ds(k * 32 + LANES, LANES)
            pr[e, sla] = a * w[e, sla]
            pr[e, slb] = b * w[e, slb]


def _sc_body(x_hbm, w_hbm, ii_hbm, ij_hbm, out_hbm,
             acc, ii0, ii1, ij0, ij1, w0, w1, xg0, xg1, pr0,
             sem_in0, sem_in1, sem_g0, sem_g1, sem_s0, sem_s1):
    cid = lax.axis_index("c")
    sid = lax.axis_index("s")
    ii = (ii0, ii1)
    ij = (ij0, ij1)
    w = (w0, w1)
    xg = (xg0, xg1)
    pr = (pr0, pr0)
    sem_in = (sem_in0, sem_in1)
    sem_g = (sem_g0, sem_g1)
    sem_s = (sem_s0, sem_s1)

    g0 = (cid * NS + sid) * EDGES_PER_TILE
    r0 = sid * ROWS_PER_TILE

    def inputs_start(c, p):
        e0 = g0 + c * CHUNK
        pltpu.async_copy(ij_hbm.at[pl.ds(e0, CHUNK)], ij[p], sem_in[p])
        pltpu.async_copy(ii_hbm.at[pl.ds(e0, CHUNK)], ii[p], sem_in[p])
        pltpu.async_copy(w_hbm.at[pl.ds(e0, CHUNK)], w[p], sem_in[p])

    def inputs_wait(p):
        pltpu.make_async_copy(ij_hbm.at[pl.ds(0, CHUNK)], ij[p], sem_in[p]).wait()
        pltpu.make_async_copy(ii_hbm.at[pl.ds(0, CHUNK)], ii[p], sem_in[p]).wait()
        pltpu.make_async_copy(w_hbm.at[pl.ds(0, CHUNK)], w[p], sem_in[p]).wait()

    def gather_start(p):
        pltpu.async_copy(x_hbm.at[ij[p]], xg[p], sem_g[p])

    def gather_wait(p):
        pltpu.make_async_copy(x_hbm.at[ij[p]], xg[p], sem_g[p]).wait()

    def scatter_start(p):
        pltpu.async_copy(pr[p], acc.at[ii[p]], sem_s[p], add=True)

    def scatter_wait(p):
        pltpu.make_async_copy(pr[p], acc.at[ii[p]], sem_s[p]).wait()

    def step(c, p, first=False, drain_only=False):
        if not drain_only:
            inputs_wait(1 - p)
            gather_start(1 - p)
        gather_wait(p)
        if not first:
            scatter_wait(1 - p)
        _mul_rows(pr[p], xg[p], w[p])
        scatter_start(p)

    inputs_start(0, 0)
    inputs_start(1, 1)

    def zbody(r, _):
        for k in range(VPR):
            pr0[r, pl.ds(k * LANES, LANES)] = jnp.zeros((LANES,), jnp.float32)
        return 0
    lax.fori_loop(0, CHUNK, zbody, 0)
    for b in range(ROWS_PER_TILE // CHUNK):
        pltpu.sync_copy(pr0, acc.at[pl.ds(r0 + b * CHUNK, CHUNK)])

    inputs_wait(0)
    gather_start(0)
    plsc.subcore_barrier()
    step(0, 0, first=True)
    inputs_start(2, 0)

    def pair(cc, _):
        c = 1 + 2 * cc
        step(c, 1)
        inputs_start(c + 2, 1)
        step(c + 1, 0)
        inputs_start(c + 3, 0)
        return 0
    lax.fori_loop(0, (NCHUNKS - 3) // 2, pair, 0)

    step(NCHUNKS - 2, 1)
    step(NCHUNKS - 1, 0, drain_only=True)
    scatter_wait(0)

    plsc.subcore_barrier()

    @pl.when(sid < NS - 1)
    def _():
        pltpu.sync_copy(acc.at[pl.ds(r0, ROWS_PER_TILE)],
                        out_hbm.at[cid, pl.ds(r0, ROWS_PER_TILE)])

    @pl.when(sid == NS - 1)
    def _():
        last = N_NODES - (NS - 1) * ROWS_PER_TILE
        pltpu.sync_copy(acc.at[pl.ds((NS - 1) * ROWS_PER_TILE, last)],
                        out_hbm.at[cid, pl.ds((NS - 1) * ROWS_PER_TILE, last)])


@jax.jit
def _cfconv_sc(x, w, ii, ij):
    mesh = plsc.VectorSubcoreMesh(core_axis_name="c", subcore_axis_name="s")
    f = pl.kernel(
        _sc_body,
        out_type=jax.ShapeDtypeStruct((NC, N_NODES, D), jnp.float32),
        mesh=mesh,
        scratch_types=[
            pltpu.VMEM_SHARED((ACC_ROWS, D), jnp.float32),
            pltpu.VMEM((CHUNK,), jnp.int32),
            pltpu.VMEM((CHUNK,), jnp.int32),
            pltpu.VMEM((CHUNK,), jnp.int32),
            pltpu.VMEM((CHUNK,), jnp.int32),
            pltpu.VMEM((CHUNK, D), jnp.float32),
            pltpu.VMEM((CHUNK, D), jnp.float32),
            pltpu.VMEM((CHUNK, D // 2), jnp.int32),
            pltpu.VMEM((CHUNK, D // 2), jnp.int32),
            pltpu.VMEM((CHUNK, D), jnp.float32),
            pltpu.SemaphoreType.DMA,
            pltpu.SemaphoreType.DMA,
            pltpu.SemaphoreType.DMA,
            pltpu.SemaphoreType.DMA,
            pltpu.SemaphoreType.DMA,
            pltpu.SemaphoreType.DMA,
        ],
        compiler_params=pltpu.CompilerParams(use_tc_tiling_on_sc=False),
    )
    return f(x, w, ii, ij)


def _add_body(a_ref, b_ref, o_ref):
    o_ref[...] = a_ref[...] + b_ref[...]


@jax.jit
def _sum_partials(p):
    blk = 1000
    return pl.pallas_call(
        _add_body,
        out_shape=jax.ShapeDtypeStruct((N_NODES, D), jnp.float32),
        grid=(N_NODES // blk,),
        in_specs=[pl.BlockSpec((blk, D), lambda i: (i, 0))] * 2,
        out_specs=pl.BlockSpec((blk, D), lambda i: (i, 0)),
    )(p[0], p[1])


_P = np.empty((D,), np.int32)
for _k in range(D // 32):
    for _l in range(16):
        _P[32 * _k + 2 * _l] = 32 * _k + _l
        _P[32 * _k + 2 * _l + 1] = 32 * _k + 16 + _l


def kernel(x, Wij, idx_i, idx_j):
    ii = idx_i.astype(jnp.int32)
    ij = idx_j.astype(jnp.int32)
    xp = lax.bitcast_convert_type(
        x[:, _P].astype(jnp.bfloat16).reshape(N_NODES, D // 2, 2), jnp.int32)
    partials = _cfconv_sc(xp, Wij, ii, ij)
    return _sum_partials(partials)


# speedup vs baseline: 1.3365x; 1.3365x over previous
import jax
import jax.numpy as jnp
from jax import lax
from jax.experimental import pallas as pl
from jax.experimental.pallas import tpu as pltpu
from jax.experimental.pallas import tpu_sc as plsc

N_NODES = 10000
N_EDGES = 320000
D = 128

NC = 2
NS = 16
LANES = 16
VPR = D // LANES

EDGES_PER_TILE = N_EDGES // (NC * NS)
CHUNK = 80
NCHUNKS = EDGES_PER_TILE // CHUNK

ACC_ROWS = 10240
ROWS_PER_TILE = ACC_ROWS // NS


def _mul_rows(xg, w):
    @plsc.parallel_loop(0, CHUNK, step=1, unroll=4)
    def _(e):
        for k in range(VPR):
            sl = pl.ds(k * LANES, LANES)
            xg[e, sl] = xg[e, sl] * w[e, sl]


def _sc_body(x_hbm, w_hbm, ii_hbm, ij_hbm, out_hbm,
             acc, ii0, ii1, ij0, ij1, w0, w1, xg0, xg1,
             sem_i0, sem_i1, sem_j0, sem_j1,
             sem_w0, sem_w1, sem_g0, sem_g1, sem_s0, sem_s1):
    cid = lax.axis_index("c")
    sid = lax.axis_index("s")
    ii = (ii0, ii1)
    ij = (ij0, ij1)
    w = (w0, w1)
    xg = (xg0, xg1)
    sem_i = (sem_i0, sem_i1)
    sem_j = (sem_j0, sem_j1)
    sem_w = (sem_w0, sem_w1)
    sem_g = (sem_g0, sem_g1)
    sem_s = (sem_s0, sem_s1)

    g0 = (cid * NS + sid) * EDGES_PER_TILE
    r0 = sid * ROWS_PER_TILE

    def ij_start(c, p):
        e0 = g0 + c * CHUNK
        pltpu.async_copy(ij_hbm.at[pl.ds(e0, CHUNK)], ij[p], sem_j[p])

    def ij_wait(p):
        pltpu.make_async_copy(ij_hbm.at[pl.ds(0, CHUNK)], ij[p], sem_j[p]).wait()

    def ii_start(c, p):
        e0 = g0 + c * CHUNK
        pltpu.async_copy(ii_hbm.at[pl.ds(e0, CHUNK)], ii[p], sem_i[p])

    def ii_wait(p):
        pltpu.make_async_copy(ii_hbm.at[pl.ds(0, CHUNK)], ii[p], sem_i[p]).wait()

    def w_start(c, p):
        e0 = g0 + c * CHUNK
        pltpu.async_copy(w_hbm.at[pl.ds(e0, CHUNK)], w[p], sem_w[p])

    def w_wait(p):
        pltpu.make_async_copy(w_hbm.at[pl.ds(0, CHUNK)], w[p], sem_w[p]).wait()

    def gather_start(p):
        pltpu.async_copy(x_hbm.at[ij[p]], xg[p], sem_g[p])

    def gather_wait(p):
        pltpu.make_async_copy(x_hbm.at[ij[p]], xg[p], sem_g[p]).wait()

    def scatter_start(p):
        pltpu.async_copy(xg[p], acc.at[ii[p]], sem_s[p], add=True)

    def scatter_wait(p):
        pltpu.make_async_copy(xg[p], acc.at[ii[p]], sem_s[p]).wait()

    def step(c, p, first=False, tail=0):
        if tail < 2:
            ij_wait(1 - p)
            if not first:
                scatter_wait(1 - p)
            gather_start(1 - p)
        gather_wait(p)
        if tail == 0:
            ij_start(c + 2, p)
        w_wait(p)
        _mul_rows(xg[p], w[p])
        ii_wait(p)
        scatter_start(p)
        if tail == 0:
            ii_start(c + 2, p)
            w_start(c + 2, p)

    ij_start(0, 0)
    ii_start(0, 0)
    w_start(0, 0)
    ij_start(1, 1)
    ii_start(1, 1)
    w_start(1, 1)

    def zbody(r, _):
        for k in range(VPR):
            xg0[r, pl.ds(k * LANES, LANES)] = jnp.zeros((LANES,), jnp.float32)
        return 0
    lax.fori_loop(0, CHUNK, zbody, 0)
    for b in range(ROWS_PER_TILE // CHUNK):
        pltpu.sync_copy(xg0, acc.at[pl.ds(r0 + b * CHUNK, CHUNK)])

    ij_wait(0)
    gather_start(0)
    plsc.subcore_barrier()
    step(0, 0, first=True)

    def pair(cc, _):
        c = 1 + 2 * cc
        step(c, 1)
        step(c + 1, 0)
        return 0
    lax.fori_loop(0, (NCHUNKS - 3) // 2, pair, 0)

    step(NCHUNKS - 2, 1, tail=1)
    step(NCHUNKS - 1, 0, tail=2)
    scatter_wait(1)
    scatter_wait(0)

    plsc.subcore_barrier()

    @pl.when(sid < NS - 1)
    def _():
        pltpu.sync_copy(acc.at[pl.ds(r0, ROWS_PER_TILE)],
                        out_hbm.at[cid, pl.ds(r0, ROWS_PER_TILE)])

    @pl.when(sid == NS - 1)
    def _():
        last = N_NODES - (NS - 1) * ROWS_PER_TILE
        pltpu.sync_copy(acc.at[pl.ds((NS - 1) * ROWS_PER_TILE, last)],
                        out_hbm.at[cid, pl.ds((NS - 1) * ROWS_PER_TILE, last)])


@jax.jit
def _cfconv_sc(x, w, ii, ij):
    mesh = plsc.VectorSubcoreMesh(core_axis_name="c", subcore_axis_name="s")
    f = pl.kernel(
        _sc_body,
        out_type=jax.ShapeDtypeStruct((NC, N_NODES, D), jnp.float32),
        mesh=mesh,
        scratch_types=[
            pltpu.VMEM_SHARED((ACC_ROWS, D), jnp.float32),
            pltpu.VMEM((CHUNK,), jnp.int32),
            pltpu.VMEM((CHUNK,), jnp.int32),
            pltpu.VMEM((CHUNK,), jnp.int32),
            pltpu.VMEM((CHUNK,), jnp.int32),
            pltpu.VMEM((CHUNK, D), jnp.float32),
            pltpu.VMEM((CHUNK, D), jnp.float32),
            pltpu.VMEM((CHUNK, D), jnp.float32),
            pltpu.VMEM((CHUNK, D), jnp.float32),
            pltpu.SemaphoreType.DMA,
            pltpu.SemaphoreType.DMA,
            pltpu.SemaphoreType.DMA,
            pltpu.SemaphoreType.DMA,
            pltpu.SemaphoreType.DMA,
            pltpu.SemaphoreType.DMA,
            pltpu.SemaphoreType.DMA,
            pltpu.SemaphoreType.DMA,
            pltpu.SemaphoreType.DMA,
            pltpu.SemaphoreType.DMA,
        ],
    )
    return f(x, w, ii, ij)


def _add_body(a_ref, b_ref, o_ref):
    o_ref[...] = a_ref[...] + b_ref[...]


@jax.jit
def _sum_partials(p):
    blk = 1000
    return pl.pallas_call(
        _add_body,
        out_shape=jax.ShapeDtypeStruct((N_NODES, D), jnp.float32),
        grid=(N_NODES // blk,),
        in_specs=[pl.BlockSpec((blk, D), lambda i: (i, 0))] * 2,
        out_specs=pl.BlockSpec((blk, D), lambda i: (i, 0)),
    )(p[0], p[1])


def kernel(x, Wij, idx_i, idx_j):
    ii = idx_i.astype(jnp.int32)
    ij = idx_j.astype(jnp.int32)
    partials = _cfconv_sc(x, Wij, ii, ij)
    return _sum_partials(partials)


# speedup vs baseline: 1.3381x; 1.0012x over previous
import jax
import jax.numpy as jnp
from jax import lax
from jax.experimental import pallas as pl
from jax.experimental.pallas import tpu as pltpu
from jax.experimental.pallas import tpu_sc as plsc

N_NODES = 10000
N_EDGES = 320000
D = 128

NC = 2
NS = 16
LANES = 16
VPR = D // LANES

EDGES_PER_TILE = N_EDGES // (NC * NS)
CHUNK = 80
NCHUNKS = EDGES_PER_TILE // CHUNK

ACC_ROWS = 10240
ROWS_PER_TILE = ACC_ROWS // NS


def _mul_rows(xg, w):
    @plsc.parallel_loop(0, CHUNK, step=1, unroll=4)
    def _(e):
        for k in range(VPR):
            sl = pl.ds(k * LANES, LANES)
            xg[e, sl] = xg[e, sl] * w[e, sl]


def _sc_body(x_hbm, x2_hbm, w_hbm, ii_hbm, ij_hbm, out_hbm,
             acc, ii0, ii1, ij0, ij1, w0, w1, xg0, xg1,
             sem_i0, sem_i1, sem_j0, sem_j1,
             sem_w0, sem_w1, sem_g0, sem_g1, sem_s0, sem_s1):
    cid = lax.axis_index("c")
    sid = lax.axis_index("s")
    ii = (ii0, ii1)
    ij = (ij0, ij1)
    w = (w0, w1)
    xg = (xg0, xg1)
    sem_i = (sem_i0, sem_i1)
    sem_j = (sem_j0, sem_j1)
    sem_w = (sem_w0, sem_w1)
    sem_g = (sem_g0, sem_g1)
    sem_s = (sem_s0, sem_s1)

    g0 = (cid * NS + sid) * EDGES_PER_TILE
    r0 = sid * ROWS_PER_TILE

    def ij_start(c, p):
        e0 = g0 + c * CHUNK
        pltpu.async_copy(ij_hbm.at[pl.ds(e0, CHUNK)], ij[p], sem_j[p])

    def ij_wait(p):
        pltpu.make_async_copy(ij_hbm.at[pl.ds(0, CHUNK)], ij[p], sem_j[p]).wait()

    def ii_start(c, p):
        e0 = g0 + c * CHUNK
        pltpu.async_copy(ii_hbm.at[pl.ds(e0, CHUNK)], ii[p], sem_i[p])

    def ii_wait(p):
        pltpu.make_async_copy(ii_hbm.at[pl.ds(0, CHUNK)], ii[p], sem_i[p]).wait()

    def w_start(c, p):
        e0 = g0 + c * CHUNK
        pltpu.async_copy(w_hbm.at[pl.ds(e0, CHUNK)], w[p], sem_w[p])

    def w_wait(p):
        pltpu.make_async_copy(w_hbm.at[pl.ds(0, CHUNK)], w[p], sem_w[p]).wait()

    def gather_start(p):
        @pl.when(cid == 0)
        def _():
            pltpu.async_copy(x_hbm.at[ij[p]], xg[p], sem_g[p])

        @pl.when(cid == 1)
        def _():
            pltpu.async_copy(x2_hbm.at[ij[p]], xg[p], sem_g[p])

    def gather_wait(p):
        pltpu.make_async_copy(x_hbm.at[ij[p]], xg[p], sem_g[p]).wait()

    def scatter_start(p):
        pltpu.async_copy(xg[p], acc.at[ii[p]], sem_s[p], add=True)

    def scatter_wait(p):
        pltpu.make_async_copy(xg[p], acc.at[ii[p]], sem_s[p]).wait()

    def step(c, p, first=False, tail=0):
        if tail < 2:
            ij_wait(1 - p)
            if not first:
                scatter_wait(1 - p)
            gather_start(1 - p)
        gather_wait(p)
        if tail == 0:
            ij_start(c + 2, p)
        w_wait(p)
        _mul_rows(xg[p], w[p])
        ii_wait(p)
        scatter_start(p)
        if tail == 0:
            ii_start(c + 2, p)
            w_start(c + 2, p)

    ij_start(0, 0)
    ii_start(0, 0)
    w_start(0, 0)
    ij_start(1, 1)
    ii_start(1, 1)
    w_start(1, 1)

    def zbody(r, _):
        for k in range(VPR):
            xg0[r, pl.ds(k * LANES, LANES)] = jnp.zeros((LANES,), jnp.float32)
        return 0
    lax.fori_loop(0, CHUNK, zbody, 0)
    for b in range(ROWS_PER_TILE // CHUNK):
        pltpu.sync_copy(xg0, acc.at[pl.ds(r0 + b * CHUNK, CHUNK)])

    ij_wait(0)
    gather_start(0)
    plsc.subcore_barrier()
    step(0, 0, first=True)

    def pair(cc, _):
        c = 1 + 2 * cc
        step(c, 1)
        step(c + 1, 0)
        return 0
    lax.fori_loop(0, (NCHUNKS - 3) // 2, pair, 0)

    step(NCHUNKS - 2, 1, tail=1)
    step(NCHUNKS - 1, 0, tail=2)
    scatter_wait(1)
    scatter_wait(0)

    plsc.subcore_barrier()

    @pl.when(sid < NS - 1)
    def _():
        pltpu.sync_copy(acc.at[pl.ds(r0, ROWS_PER_TILE)],
                        out_hbm.at[cid, pl.ds(r0, ROWS_PER_TILE)])

    @pl.when(sid == NS - 1)
    def _():
        last = N_NODES - (NS - 1) * ROWS_PER_TILE
        pltpu.sync_copy(acc.at[pl.ds((NS - 1) * ROWS_PER_TILE, last)],
                        out_hbm.at[cid, pl.ds((NS - 1) * ROWS_PER_TILE, last)])


@jax.jit
def _cfconv_sc(x, x2, w, ii, ij):
    mesh = plsc.VectorSubcoreMesh(core_axis_name="c", subcore_axis_name="s")
    f = pl.kernel(
        _sc_body,
        out_type=jax.ShapeDtypeStruct((NC, N_NODES, D), jnp.float32),
        mesh=mesh,
        scratch_types=[
            pltpu.VMEM_SHARED((ACC_ROWS, D), jnp.float32),
            pltpu.VMEM((CHUNK,), jnp.int32),
            pltpu.VMEM((CHUNK,), jnp.int32),
            pltpu.VMEM((CHUNK,), jnp.int32),
            pltpu.VMEM((CHUNK,), jnp.int32),
            pltpu.VMEM((CHUNK, D), jnp.float32),
            pltpu.VMEM((CHUNK, D), jnp.float32),
            pltpu.VMEM((CHUNK, D), jnp.float32),
            pltpu.VMEM((CHUNK, D), jnp.float32),
            pltpu.SemaphoreType.DMA,
            pltpu.SemaphoreType.DMA,
            pltpu.SemaphoreType.DMA,
            pltpu.SemaphoreType.DMA,
            pltpu.SemaphoreType.DMA,
            pltpu.SemaphoreType.DMA,
            pltpu.SemaphoreType.DMA,
            pltpu.SemaphoreType.DMA,
            pltpu.SemaphoreType.DMA,
            pltpu.SemaphoreType.DMA,
        ],
    )
    return f(x, x2, w, ii, ij)


def _add_body(a_ref, b_ref, o_ref):
    o_ref[...] = a_ref[...] + b_ref[...]


@jax.jit
def _sum_partials(p):
    blk = 1000
    return pl.pallas_call(
        _add_body,
        out_shape=jax.ShapeDtypeStruct((N_NODES, D), jnp.float32),
        grid=(N_NODES // blk,),
        in_specs=[pl.BlockSpec((blk, D), lambda i: (i, 0))] * 2,
        out_specs=pl.BlockSpec((blk, D), lambda i: (i, 0)),
    )(p[0], p[1])


def kernel(x, Wij, idx_i, idx_j):
    ii = idx_i.astype(jnp.int32)
    ij = idx_j.astype(jnp.int32)
    x2 = x + 0.0
    partials = _cfconv_sc(x, x2, Wij, ii, ij)
    return _sum_partials(partials)
